# baseline (speedup 1.0000x reference)
import functools

import jax
import jax.numpy as jnp
from jax import lax
from jax.experimental import pallas as pl
from jax.experimental.pallas import tpu as pltpu
from jax.experimental.pallas import tpu_sc as plsc

VOCAB = 1_000_000
D = 64
BATCH = 4096
HIST = 200

_INFO = plsc.get_sparse_core_info()
NC, NS = _INFO.num_cores, _INFO.num_subcores
NW = NC * NS

TOTAL = BATCH * HIST
PER_W = TOTAL // NW
CHUNK = 128
NCHUNK = PER_W // CHUNK


def _sc_gather(ids3, table128):
    mesh = plsc.VectorSubcoreMesh(core_axis_name="c", subcore_axis_name="s")

    @functools.partial(
        pl.kernel,
        mesh=mesh,
        compiler_params=pltpu.CompilerParams(use_tc_tiling_on_sc=True),
        out_type=jax.ShapeDtypeStruct((TOTAL, D), jnp.float32),
        scratch_types=[
            pltpu.VMEM((NCHUNK, CHUNK), jnp.int32),
            pltpu.VMEM((CHUNK,), jnp.int32),
            pltpu.VMEM((CHUNK, 2 * D), jnp.float32),
            pltpu.VMEM((CHUNK, D), jnp.float32),
            pltpu.SemaphoreType.DMA,
        ],
    )
    def k(ids_hbm, table_hbm, out_hbm, ids_v, idxR, stage, compact, sem_g):
        wid = lax.axis_index("s") * NC + lax.axis_index("c")
        base = wid * PER_W
        pltpu.sync_copy(ids_hbm.at[wid], ids_v)

        def step(g, _):
            for v in range(CHUNK // 16):
                ids16 = ids_v[g, pl.ds(v * 16, 16)]
                idxR[pl.ds(v * 16, 16)] = lax.shift_right_logical(ids16, 1)
            pltpu.async_copy(table_hbm.at[idxR], stage, sem_g).wait()
            for w in range(CHUNK // 16):
                ids16 = ids_v[g, pl.ds(w * 16, 16)]
                half16 = (ids16 & 1) * D
                for i in range(16):
                    j = w * 16 + i
                    half = half16[i]
                    for v in range(D // 16):
                        compact[j, pl.ds(v * 16, 16)] = stage[
                            j, pl.ds(half + v * 16, 16)
                        ]
            pltpu.sync_copy(compact, out_hbm.at[pl.ds(base + g * CHUNK, CHUNK)])
            return 0

        lax.fori_loop(0, NCHUNK, step, 0)

    return k(ids3, table128)


def kernel(input_ids, table):
    ids3 = input_ids.astype(jnp.int32).reshape(NW, NCHUNK, CHUNK)
    table128 = table.reshape(VOCAB // 2, 2 * D)
    out = _sc_gather(ids3, table128)
    return out.reshape(BATCH, HIST, D)


# speedup vs baseline: 1.2773x; 1.2773x over previous
import functools

import jax
import jax.numpy as jnp
from jax import lax
from jax.experimental import pallas as pl
from jax.experimental.pallas import tpu as pltpu
from jax.experimental.pallas import tpu_sc as plsc

VOCAB = 1_000_000
D = 64
BATCH = 4096
HIST = 200

_INFO = plsc.get_sparse_core_info()
NC, NS = _INFO.num_cores, _INFO.num_subcores
NW = NC * NS

TOTAL = BATCH * HIST
B_PER_W = BATCH // NW
PER_W = TOTAL // NW
CHUNK_A = 104
CHUNK_B = HIST - CHUNK_A
NCHUNK = 2 * BATCH // NW
K = 4
NG = NCHUNK // K


def _sc_gather(ids, table):
    mesh = plsc.VectorSubcoreMesh(core_axis_name="c", subcore_axis_name="s")

    @functools.partial(
        pl.kernel,
        mesh=mesh,
        compiler_params=pltpu.CompilerParams(use_tc_tiling_on_sc=False),
        out_type=jax.ShapeDtypeStruct((TOTAL, D), jnp.float32),
        scratch_types=[
            pltpu.VMEM((B_PER_W, HIST), jnp.int32),
            pltpu.VMEM((2 * K, CHUNK_A, D), jnp.float32),
            pltpu.SemaphoreType.DMA,
            pltpu.SemaphoreType.DMA,
            pltpu.SemaphoreType.DMA,
            pltpu.SemaphoreType.DMA,
        ],
    )
    def k(ids_hbm, table_hbm, out_hbm, idx_v, rows_v, sg0, sg1, sw0, sw1):
        wid = lax.axis_index("s") * NC + lax.axis_index("c")
        base = wid * PER_W
        pltpu.sync_copy(ids_hbm.at[pl.ds(wid * B_PER_W, B_PER_W)], idx_v)

        def sz(j):
            return CHUNK_A if j % 2 == 0 else CHUNK_B

        def off(j):
            return 0 if j % 2 == 0 else CHUNK_A

        def fire_gathers(t, set_base, sem):
            for j in range(K):
                b_local = t * (K // 2) + j // 2
                pltpu.async_copy(
                    table_hbm.at[idx_v.at[b_local, pl.ds(off(j), sz(j))]],
                    rows_v.at[set_base + j, pl.ds(0, sz(j))],
                    sem,
                )

        def drain_gathers(set_base, sem):
            for j in range(K):
                pltpu.make_async_copy(
                    table_hbm.at[idx_v.at[0, pl.ds(off(j), sz(j))]],
                    rows_v.at[set_base + j, pl.ds(0, sz(j))],
                    sem,
                ).wait()

        def fire_writes(t, set_base, sem):
            for j in range(K):
                b_local = t * (K // 2) + j // 2
                pltpu.async_copy(
                    rows_v.at[set_base + j, pl.ds(0, sz(j))],
                    out_hbm.at[pl.ds(base + b_local * HIST + off(j), sz(j))],
                    sem,
                )

        def drain_writes(set_base, sem):
            for j in range(K):
                pltpu.make_async_copy(
                    rows_v.at[set_base + j, pl.ds(0, sz(j))],
                    out_hbm.at[pl.ds(base, sz(j))],
                    sem,
                ).wait()

        fire_gathers(0, 0, sg0)

        def superstep(u, _):
            t0 = 2 * u
            drain_gathers(0, sg0)

            @pl.when(u >= 1)
            def _():
                drain_writes(K, sw1)

            fire_gathers(t0 + 1, K, sg1)
            fire_writes(t0, 0, sw0)
            drain_gathers(K, sg1)
            drain_writes(0, sw0)

            @pl.when(t0 + 2 < NG)
            def _():
                fire_gathers(t0 + 2, 0, sg0)

            fire_writes(t0 + 1, K, sw1)
            return 0

        lax.fori_loop(0, NG // 2, superstep, 0)
        drain_writes(K, sw1)

    return k(ids, table)


def kernel(input_ids, table):
    out = _sc_gather(input_ids.astype(jnp.int32), table)
    return out.reshape(BATCH, HIST, D)


# speedup vs baseline: 1.2856x; 1.0065x over previous
import functools

import jax
import jax.numpy as jnp
from jax import lax
from jax.experimental import pallas as pl
from jax.experimental.pallas import tpu as pltpu
from jax.experimental.pallas import tpu_sc as plsc

VOCAB = 1_000_000
D = 64
BATCH = 4096
HIST = 200

_INFO = plsc.get_sparse_core_info()
NC, NS = _INFO.num_cores, _INFO.num_subcores
NW = NC * NS

TOTAL = BATCH * HIST
PER_W = TOTAL // NW
CHUNK = 128
NCHUNK = PER_W // CHUNK
K = 4
NG = NCHUNK // K


def _sc_gather(ids3, table):
    mesh = plsc.VectorSubcoreMesh(core_axis_name="c", subcore_axis_name="s")

    @functools.partial(
        pl.kernel,
        mesh=mesh,
        compiler_params=pltpu.CompilerParams(use_tc_tiling_on_sc=False),
        out_type=jax.ShapeDtypeStruct((TOTAL, D), jnp.float32),
        scratch_types=[
            pltpu.VMEM((NCHUNK, CHUNK), jnp.int32),
            pltpu.VMEM((2 * K, CHUNK, D), jnp.float32),
            pltpu.SemaphoreType.DMA,
            pltpu.SemaphoreType.DMA,
            pltpu.SemaphoreType.DMA,
            pltpu.SemaphoreType.DMA,
        ],
    )
    def k(ids_hbm, table_hbm, out_hbm, idx_v, rows_v, sg0, sg1, sw0, sw1):
        wid = lax.axis_index("s") * NC + lax.axis_index("c")
        base = wid * PER_W
        pltpu.sync_copy(ids_hbm.at[wid], idx_v)

        def fire_gathers(t, set_base, sem):
            for j in range(K):
                pltpu.async_copy(
                    table_hbm.at[idx_v.at[t * K + j]], rows_v.at[set_base + j], sem
                )

        def drain_gathers(set_base, sem):
            for j in range(K):
                pltpu.make_async_copy(
                    table_hbm.at[idx_v.at[0]], rows_v.at[set_base + j], sem
                ).wait()

        def fire_writes(t, set_base, sem):
            for j in range(K):
                pltpu.async_copy(
                    rows_v.at[set_base + j],
                    out_hbm.at[pl.ds(base + (t * K + j) * CHUNK, CHUNK)],
                    sem,
                )

        def drain_writes(set_base, sem):
            for j in range(K):
                pltpu.make_async_copy(
                    rows_v.at[set_base + j], out_hbm.at[pl.ds(base, CHUNK)], sem
                ).wait()

        fire_gathers(0, 0, sg0)

        def superstep(u, _):
            t0 = 2 * u
            drain_gathers(0, sg0)

            @pl.when(u >= 1)
            def _():
                drain_writes(K, sw1)

            fire_gathers(t0 + 1, K, sg1)
            fire_writes(t0, 0, sw0)
            drain_gathers(K, sg1)
            drain_writes(0, sw0)

            @pl.when(t0 + 2 < NG)
            def _():
                fire_gathers(t0 + 2, 0, sg0)

            fire_writes(t0 + 1, K, sw1)
            return 0

        lax.fori_loop(0, NG // 2, superstep, 0)
        drain_writes(K, sw1)

    return k(ids3, table)


def kernel(input_ids, table):
    ids3 = input_ids.astype(jnp.int32).reshape(NW, NCHUNK, CHUNK)
    out = _sc_gather(ids3, table)
    return out.reshape(BATCH, HIST, D)
